# baseline (speedup 1.0000x reference)
import functools
import jax
import jax.numpy as jnp
from jax import lax
from jax.experimental import pallas as pl
from jax.experimental.pallas import tpu as pltpu
from jax.experimental.pallas import tpu_sc as plsc

_NUM_RANKS = 512
_MAX_TOKENS = 77
_DIM = 768
_C = 16
_K = 4


def _sc_body(ctx_hbm, rank_hbm, sent_hbm, out_hbm, bufs, rsems, wsems):
    nc = 2
    wid = lax.axis_index("s") * nc + lax.axis_index("c")
    nw = 32
    rpw = _NUM_RANKS // nw
    body_end = 1 + _C + _K
    tail = _MAX_TOKENS - body_end

    pltpu.sync_copy(ctx_hbm, bufs.at[0, pl.ds(1, _C)])
    pltpu.sync_copy(ctx_hbm, bufs.at[1, pl.ds(1, _C)])

    def reads(i):
        r = wid * rpw + i
        b = i % 2
        return (
            pltpu.make_async_copy(
                sent_hbm.at[r, pl.ds(0, 1)], bufs.at[b, pl.ds(0, 1)],
                rsems.at[b, 0]),
            pltpu.make_async_copy(
                rank_hbm.at[r], bufs.at[b, pl.ds(1 + _C, _K)],
                rsems.at[b, 1]),
            pltpu.make_async_copy(
                sent_hbm.at[r, pl.ds(body_end, tail)],
                bufs.at[b, pl.ds(body_end, tail)],
                rsems.at[b, 2]),
        )

    def write(i):
        r = wid * rpw + i
        b = i % 2
        return pltpu.make_async_copy(bufs.at[b], out_hbm.at[r], wsems.at[b])

    for i in range(rpw):
        if i >= 2:
            write(i - 2).wait()
        for cp in reads(i):
            cp.start()
        if i >= 1:
            for cp in reads(i - 1):
                cp.wait()
            write(i - 1).start()
    for cp in reads(rpw - 1):
        cp.wait()
    write(rpw - 1).start()
    write(rpw - 2).wait()
    write(rpw - 1).wait()


def kernel(context_embeds, rank_embeds, sentence_embeds):
    mesh = plsc.VectorSubcoreMesh(core_axis_name="c", subcore_axis_name="s")
    k = functools.partial(
        pl.kernel,
        mesh=mesh,
        out_type=jax.ShapeDtypeStruct(
            (_NUM_RANKS, _MAX_TOKENS, _DIM), sentence_embeds.dtype),
        scratch_types=[
            pltpu.VMEM((2, _MAX_TOKENS, _DIM), sentence_embeds.dtype),
            pltpu.SemaphoreType.DMA((2, 3)),
            pltpu.SemaphoreType.DMA((2,)),
        ],
        compiler_params=pltpu.CompilerParams(use_tc_tiling_on_sc=False),
    )(_sc_body)
    return k(context_embeds, rank_embeds, sentence_embeds)


# speedup vs baseline: 1.5949x; 1.5949x over previous
import functools
import jax
import jax.numpy as jnp
from jax import lax
from jax.experimental import pallas as pl
from jax.experimental.pallas import tpu as pltpu
from jax.experimental.pallas import tpu_sc as plsc

_NUM_RANKS = 512
_MAX_TOKENS = 77
_DIM = 768
_C = 16
_K = 4
_LANES = 16
_TAIL = _MAX_TOKENS - _C


def _row_copy(dst, dst_row, src, src_row, n_rows):
    for dr in range(n_rows):
        for j in range(_DIM // _LANES):
            sl = pl.ds(j * _LANES, _LANES)
            dst[dst_row + dr, sl] = src[src_row + dr, sl]


def _sc_body(ctx_hbm, rank_hbm, sent_hbm, out_hbm,
             hbuf, tbufs, c8, rbuf, zbuf, rsems, wsems):
    nc = 2
    wid = lax.axis_index("s") * nc + lax.axis_index("c")
    rpw = _NUM_RANKS // 32
    base = wid * rpw

    pltpu.sync_copy(ctx_hbm.at[pl.ds(0, 8)], zbuf)
    pltpu.sync_copy(ctx_hbm.at[pl.ds(8, 8)], c8)
    _row_copy(hbuf, 1, zbuf, 0, 8)
    _row_copy(hbuf, 9, c8, 0, 7)

    def tail_read(r, b):
        return pltpu.make_async_copy(
            sent_hbm.at[r, pl.ds(_C, _TAIL)], tbufs.at[b, pl.ds(0, _TAIL)],
            rsems.at[b, 0])

    def small_reads(r):
        return (
            pltpu.make_async_copy(
                sent_hbm.at[r, pl.ds(0, 1)], zbuf.at[pl.ds(0, 1)],
                rsems.at[0, 1]),
            pltpu.make_async_copy(rank_hbm.at[r], rbuf, rsems.at[0, 2]),
        )

    def head_write(r):
        return pltpu.make_async_copy(
            hbuf, out_hbm.at[r, pl.ds(0, _C)], wsems.at[0])

    def tail_write(r, b):
        return pltpu.make_async_copy(
            tbufs.at[b, pl.ds(0, _TAIL)], out_hbm.at[r, pl.ds(_C, _TAIL)],
            wsems.at[1 + b])

    def patch_and_write(r, b):
        _row_copy(hbuf, 0, zbuf, 0, 1)
        _row_copy(tbufs.at[b], 0, c8, 7, 1)
        _row_copy(tbufs.at[b], 1, rbuf, 0, _K)
        head_write(r).start()
        tail_write(r, b).start()

    def pair(j, carry):
        r0 = base + 2 * j
        for b in range(2):
            @pl.when(j > 0)
            def _():
                tail_write(r0 - 2 + b, b).wait()
            tail_read(r0 + b, b).start()

        for cp in small_reads(r0):
            cp.start()
        tail_read(r0, 0).wait()
        for cp in small_reads(r0):
            cp.wait()

        @pl.when(j > 0)
        def _():
            head_write(r0 - 1).wait()
        patch_and_write(r0, 0)

        for cp in small_reads(r0 + 1):
            cp.start()
        tail_read(r0 + 1, 1).wait()
        for cp in small_reads(r0 + 1):
            cp.wait()
        head_write(r0).wait()
        patch_and_write(r0 + 1, 1)
        return carry

    lax.fori_loop(0, rpw // 2, pair, 0)
    head_write(base + rpw - 1).wait()
    for b in range(2):
        tail_write(base + rpw - 2 + b, b).wait()


def kernel(context_embeds, rank_embeds, sentence_embeds):
    dt = sentence_embeds.dtype
    mesh = plsc.VectorSubcoreMesh(core_axis_name="c", subcore_axis_name="s")
    k = functools.partial(
        pl.kernel,
        mesh=mesh,
        out_type=jax.ShapeDtypeStruct((_NUM_RANKS, _MAX_TOKENS, _DIM), dt),
        scratch_types=[
            pltpu.VMEM((_C, _DIM), dt),
            pltpu.VMEM((2, _TAIL, _DIM), dt),
            pltpu.VMEM((8, _DIM), dt),
            pltpu.VMEM((_K, _DIM), dt),
            pltpu.VMEM((8, _DIM), dt),
            pltpu.SemaphoreType.DMA((2, 3)),
            pltpu.SemaphoreType.DMA((3,)),
        ],
    )(_sc_body)
    return k(context_embeds, rank_embeds, sentence_embeds)


# speedup vs baseline: 4.1202x; 2.5833x over previous
import functools
import jax
import jax.numpy as jnp
from jax import lax
from jax.experimental import pallas as pl
from jax.experimental.pallas import tpu as pltpu
from jax.experimental.pallas import tpu_sc as plsc

_NUM_RANKS = 512
_MAX_TOKENS = 77
_DIM = 768
_C = 16
_K = 4
_NW = 32
_RB = _NUM_RANKS // _NW
_CTX_REP = 8
_NRING = 3


def _sc_body(cbc_hbm, rankt_hbm, sentt_hbm, out_hbm, bufs, bbuf,
             rsems, wsems, csem, cwsem):
    nc = 2
    wid = lax.axis_index("s") * nc + lax.axis_index("c")
    ctx_row = wid % _C
    half = wid // _C
    rb = pl.ds(wid * _RB, _RB)

    cread = pltpu.make_async_copy(cbc_hbm.at[ctx_row], bbuf, csem)
    cread.start()

    rows = [0] + list(range(1 + _C, _MAX_TOKENS))

    def src(t):
        if t == 0:
            return sentt_hbm.at[0, rb]
        if t < 1 + _C + _K:
            return rankt_hbm.at[t - 1 - _C, rb]
        return sentt_hbm.at[t, rb]

    def read(n):
        return pltpu.make_async_copy(
            src(rows[n]), bufs.at[n % _NRING], rsems.at[n % _NRING])

    def write(n):
        return pltpu.make_async_copy(
            bufs.at[n % _NRING], out_hbm.at[rows[n], rb],
            wsems.at[n % _NRING])

    cread.wait()
    n_fan = _NUM_RANKS // 2 // _CTX_REP
    cwrites = [
        pltpu.make_async_copy(
            bbuf,
            out_hbm.at[1 + ctx_row,
                       pl.ds(half * (_NUM_RANKS // 2) + k * _CTX_REP,
                             _CTX_REP)],
            cwsem)
        for k in range(n_fan)
    ]
    for cw in cwrites:
        cw.start()

    n_rows = len(rows)
    for n in range(n_rows):
        if n >= _NRING:
            write(n - _NRING).wait()
        read(n).start()
        if n >= 1:
            read(n - 1).wait()
            write(n - 1).start()
    read(n_rows - 1).wait()
    write(n_rows - 1).start()
    for n in range(n_rows - _NRING, n_rows):
        write(n).wait()
    for cw in cwrites:
        cw.wait()


def kernel(context_embeds, rank_embeds, sentence_embeds):
    dt = sentence_embeds.dtype
    sent_t = jnp.transpose(sentence_embeds, (1, 0, 2))
    rank_t = jnp.transpose(rank_embeds, (1, 0, 2))
    cbc = jnp.broadcast_to(
        context_embeds[:, None, :], (_C, _CTX_REP, _DIM))
    mesh = plsc.VectorSubcoreMesh(core_axis_name="c", subcore_axis_name="s")
    k = functools.partial(
        pl.kernel,
        mesh=mesh,
        out_type=jax.ShapeDtypeStruct((_MAX_TOKENS, _NUM_RANKS, _DIM), dt),
        scratch_types=[
            pltpu.VMEM((_NRING, _RB, _DIM), dt),
            pltpu.VMEM((_CTX_REP, _DIM), dt),
            pltpu.SemaphoreType.DMA((_NRING,)),
            pltpu.SemaphoreType.DMA((_NRING,)),
            pltpu.SemaphoreType.DMA,
            pltpu.SemaphoreType.DMA,
        ],
    )(_sc_body)
    out_t = k(cbc, rank_t, sent_t)
    return jnp.transpose(out_t, (1, 0, 2))


# speedup vs baseline: 4.2772x; 1.0381x over previous
import functools
import jax
import jax.numpy as jnp
from jax import lax
from jax.experimental import pallas as pl
from jax.experimental.pallas import tpu as pltpu
from jax.experimental.pallas import tpu_sc as plsc

_NUM_RANKS = 512
_MAX_TOKENS = 77
_DIM = 768
_C = 16
_K = 4
_NW = 32
_RB = _NUM_RANKS // _NW
_CTX_REP = 16
_NRING = 3


def _sc_body(cbc_hbm, rankt_hbm, sentt_hbm, out_hbm, bufs, bbuf,
             rsems, wsems, csem, cwsem):
    nc = 2
    wid = lax.axis_index("s") * nc + lax.axis_index("c")
    ctx_row = wid % _C
    half = wid // _C
    rb = pl.ds(wid * _RB, _RB)

    cread = pltpu.make_async_copy(cbc_hbm.at[ctx_row], bbuf, csem)
    cread.start()

    chunks = [(0, 1)] + [(1 + _C, 2), (3 + _C, 2)]
    chunks += [(t, 2) for t in range(1 + _C + _K, _MAX_TOKENS, 2)]

    def src(t, m):
        if t == 0:
            return sentt_hbm.at[pl.ds(0, m), rb]
        if t < 1 + _C + _K:
            return rankt_hbm.at[pl.ds(t - 1 - _C, m), rb]
        return sentt_hbm.at[pl.ds(t, m), rb]

    def read(n):
        t, m = chunks[n]
        return pltpu.make_async_copy(
            src(t, m), bufs.at[n % _NRING, pl.ds(0, m)],
            rsems.at[n % _NRING])

    def write(n):
        t, m = chunks[n]
        return pltpu.make_async_copy(
            bufs.at[n % _NRING, pl.ds(0, m)],
            out_hbm.at[pl.ds(t, m), rb],
            wsems.at[n % _NRING])

    cread.wait()
    n_fan = _NUM_RANKS // 2 // _CTX_REP
    cwrites = [
        pltpu.make_async_copy(
            bbuf,
            out_hbm.at[1 + ctx_row,
                       pl.ds(half * (_NUM_RANKS // 2) + k * _CTX_REP,
                             _CTX_REP)],
            cwsem)
        for k in range(n_fan)
    ]
    for cw in cwrites:
        cw.start()

    n_rows = len(chunks)
    for n in range(n_rows):
        if n >= _NRING:
            write(n - _NRING).wait()
        read(n).start()
        if n >= 1:
            read(n - 1).wait()
            write(n - 1).start()
    read(n_rows - 1).wait()
    write(n_rows - 1).start()
    for n in range(n_rows - _NRING, n_rows):
        write(n).wait()
    for cw in cwrites:
        cw.wait()


def kernel(context_embeds, rank_embeds, sentence_embeds):
    dt = sentence_embeds.dtype
    sent_t = jnp.transpose(sentence_embeds, (1, 0, 2))
    rank_t = jnp.transpose(rank_embeds, (1, 0, 2))
    cbc = jnp.broadcast_to(
        context_embeds[:, None, :], (_C, _CTX_REP, _DIM))
    mesh = plsc.VectorSubcoreMesh(core_axis_name="c", subcore_axis_name="s")
    k = functools.partial(
        pl.kernel,
        mesh=mesh,
        out_type=jax.ShapeDtypeStruct((_MAX_TOKENS, _NUM_RANKS, _DIM), dt),
        scratch_types=[
            pltpu.VMEM((_NRING, 2, _RB, _DIM), dt),
            pltpu.VMEM((_CTX_REP, _DIM), dt),
            pltpu.SemaphoreType.DMA((_NRING,)),
            pltpu.SemaphoreType.DMA((_NRING,)),
            pltpu.SemaphoreType.DMA,
            pltpu.SemaphoreType.DMA,
        ],
    )(_sc_body)
    out_t = k(cbc, rank_t, sent_t)
    return jnp.transpose(out_t, (1, 0, 2))


# speedup vs baseline: 4.3114x; 1.0080x over previous
import functools
import jax
import jax.numpy as jnp
from jax import lax
from jax.experimental import pallas as pl
from jax.experimental.pallas import tpu as pltpu
from jax.experimental.pallas import tpu_sc as plsc

_NUM_RANKS = 512
_MAX_TOKENS = 77
_DIM = 768
_C = 16
_K = 4
_NW = 32
_RB = _NUM_RANKS // _NW
_CTX_REP = 16
_NRING = 4


def _sc_body(cbc_hbm, rankt_hbm, sentt_hbm, out_hbm, bufs, bbuf,
             rsems, wsems, csem, cwsem):
    nc = 2
    wid = lax.axis_index("s") * nc + lax.axis_index("c")
    ctx_row = wid % _C
    half = wid // _C
    rb = pl.ds(wid * _RB, _RB)

    cread = pltpu.make_async_copy(cbc_hbm.at[ctx_row], bbuf, csem)
    cread.start()

    chunks = [(0, 1)] + [(1 + _C, 2), (3 + _C, 2)]
    chunks += [(t, 2) for t in range(1 + _C + _K, _MAX_TOKENS, 2)]

    def src(t, m):
        if t == 0:
            return sentt_hbm.at[pl.ds(0, m), rb]
        if t < 1 + _C + _K:
            return rankt_hbm.at[pl.ds(t - 1 - _C, m), rb]
        return sentt_hbm.at[pl.ds(t, m), rb]

    def read(n):
        t, m = chunks[n]
        return pltpu.make_async_copy(
            src(t, m), bufs.at[n % _NRING, pl.ds(0, m)],
            rsems.at[n % _NRING])

    def write(n):
        t, m = chunks[n]
        return pltpu.make_async_copy(
            bufs.at[n % _NRING, pl.ds(0, m)],
            out_hbm.at[pl.ds(t, m), rb],
            wsems.at[n % _NRING])

    cread.wait()
    n_fan = _NUM_RANKS // 2 // _CTX_REP
    cwrites = [
        pltpu.make_async_copy(
            bbuf,
            out_hbm.at[1 + ctx_row,
                       pl.ds(half * (_NUM_RANKS // 2) + k * _CTX_REP,
                             _CTX_REP)],
            cwsem)
        for k in range(n_fan)
    ]
    for cw in cwrites:
        cw.start()

    n_rows = len(chunks)
    for n in range(n_rows):
        if n >= _NRING:
            write(n - _NRING).wait()
        read(n).start()
        if n >= 1:
            read(n - 1).wait()
            write(n - 1).start()
    read(n_rows - 1).wait()
    write(n_rows - 1).start()
    for n in range(n_rows - _NRING, n_rows):
        write(n).wait()
    for cw in cwrites:
        cw.wait()


def kernel(context_embeds, rank_embeds, sentence_embeds):
    dt = sentence_embeds.dtype
    sent_t = jnp.transpose(sentence_embeds, (1, 0, 2))
    rank_t = jnp.transpose(rank_embeds, (1, 0, 2))
    cbc = jnp.broadcast_to(
        context_embeds[:, None, :], (_C, _CTX_REP, _DIM))
    mesh = plsc.VectorSubcoreMesh(core_axis_name="c", subcore_axis_name="s")
    k = functools.partial(
        pl.kernel,
        mesh=mesh,
        out_type=jax.ShapeDtypeStruct((_MAX_TOKENS, _NUM_RANKS, _DIM), dt),
        scratch_types=[
            pltpu.VMEM((_NRING, 2, _RB, _DIM), dt),
            pltpu.VMEM((_CTX_REP, _DIM), dt),
            pltpu.SemaphoreType.DMA((_NRING,)),
            pltpu.SemaphoreType.DMA((_NRING,)),
            pltpu.SemaphoreType.DMA,
            pltpu.SemaphoreType.DMA,
        ],
    )(_sc_body)
    out_t = k(cbc, rank_t, sent_t)
    return jnp.transpose(out_t, (1, 0, 2))
